# baseline (speedup 1.0000x reference)
import functools
import jax
import jax.numpy as jnp
from jax import lax
from jax.experimental import pallas as pl
from jax.experimental.pallas import tpu as pltpu, tpu_sc as plsc

BATCH = 16
NUM_POINTS = 8192
TABLE_ROWS = 1024
HALF = 32
TABLE_FLAT = 2 * TABLE_ROWS * HALF

NPAIRS = BATCH * NUM_POINTS
NW = 32
PTS_PER_W = NPAIRS // NW
CHUNKP = 256
NCHUNK = PTS_PER_W // CHUNKP
NGROUP = CHUNKP // 16

_mesh = plsc.VectorSubcoreMesh(core_axis_name="c", subcore_axis_name="s")


@functools.partial(
    pl.kernel,
    out_type=jax.ShapeDtypeStruct((NPAIRS * 2 * HALF,), jnp.float32),
    mesh=_mesh,
    scratch_types=[
        pltpu.VMEM((TABLE_FLAT,), jnp.float32),
        pltpu.VMEM((2 * CHUNKP,), jnp.int32),
        pltpu.VMEM((2 * CHUNKP,), jnp.int32),
        pltpu.VMEM((CHUNKP * 2 * HALF,), jnp.float32),
        pltpu.VMEM((CHUNKP * 2 * HALF,), jnp.float32),
        pltpu.SemaphoreType.DMA,
        pltpu.SemaphoreType.DMA,
    ],
    compiler_params=pltpu.CompilerParams(
        use_tc_tiling_on_sc=False, needs_layout_passes=False
    ),
)
def _sc_lookup(coords_hbm, table_hbm, out_hbm, table_v, cidx0, cidx1, outv0, outv1, osem0, osem1):
    wid = lax.axis_index("s") * 2 + lax.axis_index("c")
    pltpu.sync_copy(table_hbm, table_v)

    iota = lax.iota(jnp.int32, 16)
    osem = (osem0, osem1)
    cidx = (cidx0, cidx1)
    outv = (outv0, outv1)
    out_handles = [None, None]

    for g in range(NCHUNK):
        b = g & 1
        p0 = wid * PTS_PER_W + g * CHUNKP
        if out_handles[b] is not None:
            out_handles[b].wait()
            out_handles[b] = None
        pltpu.sync_copy(coords_hbm.at[pl.ds(p0 * 2, 2 * CHUNKP)], cidx[b])
        cchunk = cidx[b]
        obuf = outv[b]

        def group_body(q, _):
            rel = q * 16
            xpos = rel * 2 + 2 * iota
            tx = plsc.load_gather(cchunk, [xpos])
            ty = plsc.load_gather(cchunk, [xpos + 1])
            gx = tx * HALF
            gy = ty * HALF + TABLE_ROWS * HALF
            dbase = (rel + iota) * (2 * HALF)
            for d in range(HALF):
                vx = plsc.load_gather(table_v, [gx + d])
                plsc.store_scatter(obuf, [dbase + d], vx)
                vy = plsc.load_gather(table_v, [gy + d])
                plsc.store_scatter(obuf, [dbase + HALF + d], vy)
            return 0

        lax.fori_loop(0, NGROUP, group_body, 0)
        out_handles[b] = pltpu.async_copy(
            obuf, out_hbm.at[pl.ds(p0 * 2 * HALF, CHUNKP * 2 * HALF)], osem[b]
        )

    for h in out_handles:
        if h is not None:
            h.wait()


def kernel(pixel_coordinates, x_table, y_table):
    coords = pixel_coordinates.reshape(-1)
    table = jnp.concatenate([x_table, y_table], axis=0).reshape(-1)
    out = _sc_lookup(coords, table)
    return out.reshape(BATCH, NUM_POINTS, 2 * HALF)


# speedup vs baseline: 2.5543x; 2.5543x over previous
import functools
import jax
import jax.numpy as jnp
from jax import lax
from jax.experimental import pallas as pl
from jax.experimental.pallas import tpu as pltpu, tpu_sc as plsc

BATCH = 16
NUM_POINTS = 8192
TABLE_ROWS = 1024
HALF = 32

NPAIRS = BATCH * NUM_POINTS
NROWS = 2 * NPAIRS
NW = 32
ROWS_PER_W = NROWS // NW
CHUNK = 1024
NCHUNK = ROWS_PER_W // CHUNK
GSIZE = 128
NG = CHUNK // GSIZE

_mesh = plsc.VectorSubcoreMesh(core_axis_name="c", subcore_axis_name="s")


@functools.partial(
    pl.kernel,
    out_type=jax.ShapeDtypeStruct((NROWS, HALF), jnp.float32),
    mesh=_mesh,
    scratch_types=[
        pltpu.VMEM_SHARED((2 * TABLE_ROWS, HALF), jnp.float32),
        pltpu.VMEM((2, NG, GSIZE), jnp.int32),
        pltpu.VMEM((2, CHUNK, HALF), jnp.float32),
        pltpu.SemaphoreType.DMA,
        pltpu.SemaphoreType.DMA,
        pltpu.SemaphoreType.DMA,
        pltpu.SemaphoreType.DMA,
    ],
    compiler_params=pltpu.CompilerParams(use_tc_tiling_on_sc=False),
)
def _sc_gather(coords_hbm, table_hbm, out_hbm, table_sh, idx_v, rows_v,
               gsem0, gsem1, osem0, osem1):
    wid = lax.axis_index("s") * 2 + lax.axis_index("c")
    @pl.when(lax.axis_index("s") == 0)
    def _():
        pltpu.sync_copy(table_hbm, table_sh)

    plsc.subcore_barrier()

    offs = (lax.iota(jnp.int32, 16) & 1) * TABLE_ROWS
    gsem = (gsem0, gsem1)
    osem = (osem0, osem1)

    out_handles = [None, None]
    prev = None
    for g in range(NCHUNK):
        b = g & 1
        row0 = wid * ROWS_PER_W + g * CHUNK
        if out_handles[b] is not None:
            out_handles[b].wait()
            out_handles[b] = None
        crow0 = pl.multiple_of(row0 // GSIZE, 8)
        pltpu.sync_copy(coords_hbm.at[pl.ds(crow0, NG), :], idx_v.at[b])
        for j in range(NG):
            row = idx_v.at[b, j]

            def add_off(i, _):
                sl = pl.ds(i * 16, 16)
                row[sl] = row[sl] + offs
                return 0

            lax.fori_loop(0, GSIZE // 16, add_off, 0)
        gh = [
            pltpu.async_copy(
                table_sh.at[idx_v.at[b, j]],
                rows_v.at[b, pl.ds(j * GSIZE, GSIZE), :],
                gsem[b],
            )
            for j in range(NG)
        ]
        if prev is not None:
            pb, pgh, prow0 = prev
            for c in pgh:
                c.wait()
            out_handles[pb] = pltpu.async_copy(
                rows_v.at[pb], out_hbm.at[pl.ds(prow0, CHUNK), :], osem[pb]
            )
        prev = (b, gh, row0)

    pb, pgh, prow0 = prev
    for c in pgh:
        c.wait()
    out_handles[pb] = pltpu.async_copy(
        rows_v.at[pb], out_hbm.at[pl.ds(prow0, CHUNK), :], osem[pb]
    )
    for h in out_handles:
        if h is not None:
            h.wait()


def kernel(pixel_coordinates, x_table, y_table):
    coords = pixel_coordinates.reshape(NROWS // GSIZE, GSIZE)
    table = jnp.concatenate([x_table, y_table], axis=0)
    out = _sc_gather(coords, table)
    return out.reshape(BATCH, NUM_POINTS, 2 * HALF)
